# baseline (speedup 1.0000x reference)
import jax
import jax.numpy as jnp
from jax.experimental import pallas as pl
from jax.experimental.pallas import tpu as pltpu

D_MODEL = 768
D_FF = 3072
NUM_EXPERTS = 64
Z_LOSS_COEF = 0.001
TILE = 512


def _moe_kernel(x_ref, gw_ref, uw_ref, ub_ref, dw_ref, db_ref,
                out_ref, aux_ref, acc_ref):
    i = pl.program_id(0)
    nt = pl.num_programs(0)

    @pl.when(i == 0)
    def _init():
        acc_ref[...] = jnp.zeros_like(acc_ref)

    x = x_ref[...]

    logits = jnp.dot(x, gw_ref[...], preferred_element_type=jnp.float32)
    m = jnp.max(logits, axis=-1, keepdims=True)
    e = jnp.exp(logits - m)
    probs = e / jnp.sum(e, axis=-1, keepdims=True)

    idx = jax.lax.broadcasted_iota(jnp.int32, probs.shape, 1)
    m1 = jnp.max(probs, axis=-1, keepdims=True)
    i1 = jnp.min(jnp.where(probs == m1, idx, NUM_EXPERTS), axis=-1,
                 keepdims=True)
    oh1 = idx == i1
    probs_m = jnp.where(oh1, -1.0, probs)
    m2 = jnp.max(probs_m, axis=-1, keepdims=True)
    i2 = jnp.min(jnp.where(probs_m == m2, idx, NUM_EXPERTS), axis=-1,
                 keepdims=True)
    onehot2 = (idx == i2).astype(jnp.float32)
    onehot1 = oh1.astype(jnp.float32)

    counts = jnp.sum(onehot1 + onehot2, axis=0, keepdims=True)
    imp = jnp.sum(probs, axis=0, keepdims=True)
    acc_ref[0:1, :] += counts
    acc_ref[1:2, :] += imp

    h = jnp.maximum(
        jnp.dot(x.astype(jnp.bfloat16), uw_ref[...],
                preferred_element_type=jnp.float32)
        + ub_ref[...], 0.0)
    out_ref[...] = (jnp.dot(h.astype(jnp.bfloat16), dw_ref[...],
                            preferred_element_type=jnp.float32)
                    + db_ref[...])

    @pl.when(i == nt - 1)
    def _finalize():
        n = jnp.float32(nt * TILE)
        load = acc_ref[0:1, :] / n
        impf = acc_ref[1:2, :] / n
        aux_ref[...] = (jnp.float32(NUM_EXPERTS)
                        * jnp.sum(load * impf, keepdims=True)
                        * jnp.float32(Z_LOSS_COEF))


def kernel(x, gate_w, up_w, up_b, down_w, down_b, deterministic=True):
    batch_size, seq_len, d_model = x.shape
    num_tokens = batch_size * seq_len
    x_flat = x.reshape(num_tokens, d_model)
    nt = num_tokens // TILE

    out_flat, aux = pl.pallas_call(
        _moe_kernel,
        grid=(nt,),
        in_specs=[
            pl.BlockSpec((TILE, d_model), lambda i: (i, 0)),
            pl.BlockSpec((d_model, NUM_EXPERTS), lambda i: (0, 0)),
            pl.BlockSpec((d_model, D_FF), lambda i: (0, 0)),
            pl.BlockSpec((1, D_FF), lambda i: (0, 0)),
            pl.BlockSpec((D_FF, d_model), lambda i: (0, 0)),
            pl.BlockSpec((1, d_model), lambda i: (0, 0)),
        ],
        out_specs=[
            pl.BlockSpec((TILE, d_model), lambda i: (i, 0)),
            pl.BlockSpec((1, 1), lambda i: (0, 0)),
        ],
        out_shape=[
            jax.ShapeDtypeStruct((num_tokens, d_model), jnp.float32),
            jax.ShapeDtypeStruct((1, 1), jnp.float32),
        ],
        scratch_shapes=[pltpu.VMEM((2, NUM_EXPERTS), jnp.float32)],
    )(x_flat, gate_w, up_w.astype(jnp.bfloat16), up_b.reshape(1, D_FF),
      down_w.astype(jnp.bfloat16), down_b.reshape(1, d_model))

    return out_flat.reshape(batch_size, seq_len, d_model), aux[0, 0]


# speedup vs baseline: 1.0775x; 1.0775x over previous
import jax
import jax.numpy as jnp
from jax.experimental import pallas as pl
from jax.experimental.pallas import tpu as pltpu

D_MODEL = 768
D_FF = 3072
NUM_EXPERTS = 64
Z_LOSS_COEF = 0.001
TILE = 512
FF_CHUNK = 768


def _moe_kernel(x_ref, gw_ref, uw_ref, ub_ref, dw_ref, db_ref,
                out_ref, aux_ref, acc_ref, uwb_ref, dwb_ref):
    i = pl.program_id(0)
    nt = pl.num_programs(0)

    @pl.when(i == 0)
    def _init():
        acc_ref[...] = jnp.zeros_like(acc_ref)
        uwb_ref[...] = uw_ref[...].astype(jnp.bfloat16)
        dwb_ref[...] = dw_ref[...].astype(jnp.bfloat16)

    x = x_ref[...]

    logits = jnp.dot(x, gw_ref[...], preferred_element_type=jnp.float32)
    m = jnp.max(logits, axis=-1, keepdims=True)
    e = jnp.exp(logits - m)
    probs = e / jnp.sum(e, axis=-1, keepdims=True)

    idx = jax.lax.broadcasted_iota(jnp.int32, probs.shape, 1)
    m1 = jnp.max(probs, axis=-1, keepdims=True)
    i1 = jnp.min(jnp.where(probs == m1, idx, NUM_EXPERTS), axis=-1,
                 keepdims=True)
    oh1 = idx == i1
    probs_m = jnp.where(oh1, -1.0, probs)
    m2 = jnp.max(probs_m, axis=-1, keepdims=True)
    i2 = jnp.min(jnp.where(probs_m == m2, idx, NUM_EXPERTS), axis=-1,
                 keepdims=True)
    onehot2 = (idx == i2).astype(jnp.float32)
    onehot1 = oh1.astype(jnp.float32)

    counts = jnp.sum(onehot1 + onehot2, axis=0, keepdims=True)
    imp = jnp.sum(probs, axis=0, keepdims=True)
    acc_ref[0:1, :] += counts
    acc_ref[1:2, :] += imp

    xb = x.astype(jnp.bfloat16)
    acc = db_ref[...].astype(jnp.float32)
    for c in range(D_FF // FF_CHUNK):
        sl = pl.ds(c * FF_CHUNK, FF_CHUNK)
        h = jnp.maximum(
            jnp.dot(xb, uwb_ref[:, sl], preferred_element_type=jnp.float32)
            + ub_ref[:, sl], 0.0)
        acc = acc + jnp.dot(h.astype(jnp.bfloat16), dwb_ref[sl, :],
                            preferred_element_type=jnp.float32)
    out_ref[...] = acc

    @pl.when(i == nt - 1)
    def _finalize():
        n = jnp.float32(nt * TILE)
        load = acc_ref[0:1, :] / n
        impf = acc_ref[1:2, :] / n
        aux_ref[...] = (jnp.float32(NUM_EXPERTS)
                        * jnp.sum(load * impf, keepdims=True)
                        * jnp.float32(Z_LOSS_COEF))


def kernel(x, gate_w, up_w, up_b, down_w, down_b, deterministic=True):
    batch_size, seq_len, d_model = x.shape
    num_tokens = batch_size * seq_len
    x_flat = x.reshape(num_tokens, d_model)
    nt = num_tokens // TILE

    out_flat, aux = pl.pallas_call(
        _moe_kernel,
        grid=(nt,),
        in_specs=[
            pl.BlockSpec((TILE, d_model), lambda i: (i, 0)),
            pl.BlockSpec((d_model, NUM_EXPERTS), lambda i: (0, 0)),
            pl.BlockSpec((d_model, D_FF), lambda i: (0, 0)),
            pl.BlockSpec((1, D_FF), lambda i: (0, 0)),
            pl.BlockSpec((D_FF, d_model), lambda i: (0, 0)),
            pl.BlockSpec((1, d_model), lambda i: (0, 0)),
        ],
        out_specs=[
            pl.BlockSpec((TILE, d_model), lambda i: (i, 0)),
            pl.BlockSpec((1, 1), lambda i: (0, 0)),
        ],
        out_shape=[
            jax.ShapeDtypeStruct((num_tokens, d_model), jnp.float32),
            jax.ShapeDtypeStruct((1, 1), jnp.float32),
        ],
        scratch_shapes=[
            pltpu.VMEM((2, NUM_EXPERTS), jnp.float32),
            pltpu.VMEM((D_MODEL, D_FF), jnp.bfloat16),
            pltpu.VMEM((D_FF, D_MODEL), jnp.bfloat16),
        ],
    )(x_flat, gate_w, up_w, up_b.reshape(1, D_FF), down_w,
      down_b.reshape(1, d_model))

    return out_flat.reshape(batch_size, seq_len, d_model), aux[0, 0]


# speedup vs baseline: 1.1030x; 1.0237x over previous
import jax
import jax.numpy as jnp
from jax.experimental import pallas as pl
from jax.experimental.pallas import tpu as pltpu

D_MODEL = 768
D_FF = 3072
NUM_EXPERTS = 64
Z_LOSS_COEF = 0.001
TILE = 1024
FF_CHUNK = 768


def _moe_kernel(x_ref, gw_ref, uw_ref, ub_ref, dw_ref, db_ref,
                out_ref, aux_ref, acc_ref):
    i = pl.program_id(0)
    nt = pl.num_programs(0)

    @pl.when(i == 0)
    def _init():
        acc_ref[...] = jnp.zeros_like(acc_ref)

    x = x_ref[...]

    logits = jnp.dot(x, gw_ref[...], preferred_element_type=jnp.float32)
    m = jnp.max(logits, axis=-1, keepdims=True)
    e = jnp.exp(logits - m)
    probs = e / jnp.sum(e, axis=-1, keepdims=True)

    idx = jax.lax.broadcasted_iota(jnp.int32, probs.shape, 1)
    m1 = jnp.max(probs, axis=-1, keepdims=True)
    i1 = jnp.min(jnp.where(probs == m1, idx, NUM_EXPERTS), axis=-1,
                 keepdims=True)
    oh1 = idx == i1
    probs_m = jnp.where(oh1, -1.0, probs)
    m2 = jnp.max(probs_m, axis=-1, keepdims=True)
    i2 = jnp.min(jnp.where(probs_m == m2, idx, NUM_EXPERTS), axis=-1,
                 keepdims=True)
    onehot2 = (idx == i2).astype(jnp.float32)
    onehot1 = oh1.astype(jnp.float32)

    counts = jnp.sum(onehot1 + onehot2, axis=0, keepdims=True)
    imp = jnp.sum(probs, axis=0, keepdims=True)
    acc_ref[0:1, :] += counts
    acc_ref[1:2, :] += imp

    acc = db_ref[...].astype(jnp.float32)
    for c in range(D_FF // FF_CHUNK):
        sl = pl.ds(c * FF_CHUNK, FF_CHUNK)
        h = jnp.maximum(
            jnp.dot(x, uw_ref[:, sl], preferred_element_type=jnp.float32)
            + ub_ref[:, sl], 0.0)
        acc = acc + jnp.dot(h, dw_ref[sl, :],
                            preferred_element_type=jnp.float32)
    out_ref[...] = acc

    @pl.when(i == nt - 1)
    def _finalize():
        n = jnp.float32(nt * TILE)
        load = acc_ref[0:1, :] / n
        impf = acc_ref[1:2, :] / n
        aux_ref[...] = (jnp.float32(NUM_EXPERTS)
                        * jnp.sum(load * impf, keepdims=True)
                        * jnp.float32(Z_LOSS_COEF))


def kernel(x, gate_w, up_w, up_b, down_w, down_b, deterministic=True):
    batch_size, seq_len, d_model = x.shape
    num_tokens = batch_size * seq_len
    x_flat = x.reshape(num_tokens, d_model)
    nt = num_tokens // TILE

    out_flat, aux = pl.pallas_call(
        _moe_kernel,
        grid=(nt,),
        in_specs=[
            pl.BlockSpec((TILE, d_model), lambda i: (i, 0)),
            pl.BlockSpec((d_model, NUM_EXPERTS), lambda i: (0, 0)),
            pl.BlockSpec((d_model, D_FF), lambda i: (0, 0)),
            pl.BlockSpec((1, D_FF), lambda i: (0, 0)),
            pl.BlockSpec((D_FF, d_model), lambda i: (0, 0)),
            pl.BlockSpec((1, d_model), lambda i: (0, 0)),
        ],
        out_specs=[
            pl.BlockSpec((TILE, d_model), lambda i: (i, 0)),
            pl.BlockSpec((1, 1), lambda i: (0, 0)),
        ],
        out_shape=[
            jax.ShapeDtypeStruct((num_tokens, d_model), jnp.float32),
            jax.ShapeDtypeStruct((1, 1), jnp.float32),
        ],
        scratch_shapes=[pltpu.VMEM((2, NUM_EXPERTS), jnp.float32)],
    )(x_flat, gate_w, up_w, up_b.reshape(1, D_FF), down_w,
      down_b.reshape(1, d_model))

    return out_flat.reshape(batch_size, seq_len, d_model), aux[0, 0]
